# baseline (speedup 1.0000x reference)
import functools

import jax
import jax.numpy as jnp
from jax import lax
from jax.experimental import pallas as pl
from jax.experimental.pallas import tpu as pltpu
from jax.experimental.pallas import tpu_sc as plsc

N_NODES = 10000
N_EDGES = 160000
D_IN = 256
D_HALF = 128

NTILES = 16
NCORES = 2
R = 10240
ROWS_PER_TILE = R // NTILES
ABSORB = N_NODES
CH = 128
NCH = 80
QC = 40
NBUF = 2
E_PAD = NTILES * NCH * CH
ROW_CH = 128


def _sc_agg_body(table, srcs, dsts, agg_out, *scratch):
    src_a, dst_a = scratch[0], scratch[1]
    rows = scratch[2:2 + NBUF]
    gs = scratch[2 + NBUF:2 + 2 * NBUF]
    s0 = scratch[2 + 2 * NBUF]
    agg_s = scratch[3 + 2 * NBUF]
    rows0 = rows[0]
    c = lax.axis_index("c")
    s = lax.axis_index("s")
    base = s * ROWS_PER_TILE

    def _zrow(k, _):
        rows0[k // 8, pl.ds((k % 8) * 16, 16)] = jnp.zeros((16,), jnp.float32)
        return 0

    lax.fori_loop(0, CH * 8, _zrow, 0)
    for k in range(ROWS_PER_TILE // CH):
        pltpu.sync_copy(rows0, agg_s.at[pl.ds(base + k * CH, CH)])

    plsc.subcore_barrier()

    for q in range(NCH // QC):
        pltpu.sync_copy(srcs.at[c, s, pl.ds(q * QC, QC)], src_a)
        pltpu.sync_copy(dsts.at[s, pl.ds(q * QC, QC)], dst_a)
        for b in range(NBUF):
            pltpu.async_copy(table.at[src_a.at[b]], rows[b], gs[b])

        def _grp(k, _):
            jj = NBUF * k
            for b in range(NBUF):
                pltpu.make_async_copy(table.at[src_a.at[jj + b]], rows[b], gs[b]).wait()
                pltpu.async_copy(rows[b], agg_s.at[dst_a.at[jj + b]], s0, add=True)
                pltpu.make_async_copy(rows[b], agg_s.at[dst_a.at[jj + b]], s0).wait()

                def _issue(b=b, jj=jj):
                    pltpu.async_copy(table.at[src_a.at[jj + b + NBUF]], rows[b], gs[b])

                pl.when(k < QC // NBUF - 1)(_issue)
            return 0

        lax.fori_loop(0, QC // NBUF, _grp, 0)
    plsc.subcore_barrier()

    pltpu.sync_copy(agg_s.at[pl.ds(base, ROWS_PER_TILE)],
                    agg_out.at[c, pl.ds(base, ROWS_PER_TILE)])


def _make_sc_agg():
    mesh = plsc.VectorSubcoreMesh(core_axis_name="c", subcore_axis_name="s")
    return pl.kernel(
        _sc_agg_body,
        out_type=jax.ShapeDtypeStruct((NCORES, R, D_HALF), jnp.float32),
        mesh=mesh,
        scratch_types=(
            [pltpu.VMEM((QC, CH), jnp.int32),
             pltpu.VMEM((QC, CH), jnp.int32)]
            + [pltpu.VMEM((CH, D_HALF), jnp.float32) for _ in range(NBUF)]
            + [pltpu.SemaphoreType.DMA for _ in range(NBUF + 1)]
            + [pltpu.VMEM_SHARED((R, D_HALF), jnp.float32)]
        ),
    )


def _sc_deg_body(dsts, deg_out, dst_a, ones_v, sem, deg_s):
    c = lax.axis_index("c")
    s = lax.axis_index("s")
    base = s * ROWS_PER_TILE
    nch = NCH // 2

    def _fill(v):
        def _row(k, _):
            ones_v[k // 8, pl.ds((k % 8) * 16, 16)] = jnp.full((16,), v, jnp.float32)
            return 0

        lax.fori_loop(0, CH * 8, _row, 0)

    _fill(0.0)
    for k in range(ROWS_PER_TILE // CH):
        pltpu.sync_copy(ones_v, deg_s.at[pl.ds(base + k * CH, CH)])
    _fill(1.0)
    pltpu.sync_copy(dsts.at[s, pl.ds(c * nch, nch)], dst_a)
    plsc.subcore_barrier()

    def _chunk(j, _):
        pltpu.async_copy(ones_v, deg_s.at[dst_a.at[j]], sem, add=True)
        pltpu.make_async_copy(ones_v, deg_s.at[dst_a.at[j]], sem).wait()
        return 0

    lax.fori_loop(0, nch, _chunk, 0)
    plsc.subcore_barrier()

    pltpu.sync_copy(deg_s.at[pl.ds(base, ROWS_PER_TILE)],
                    deg_out.at[c, pl.ds(base, ROWS_PER_TILE)])


def _make_sc_deg():
    mesh = plsc.VectorSubcoreMesh(core_axis_name="c", subcore_axis_name="s")
    return pl.kernel(
        _sc_deg_body,
        out_type=jax.ShapeDtypeStruct((NCORES, R, D_HALF), jnp.float32),
        mesh=mesh,
        scratch_types=[
            pltpu.VMEM((NCH // 2, CH), jnp.int32),
            pltpu.VMEM((CH, D_HALF), jnp.float32),
            pltpu.SemaphoreType.DMA,
            pltpu.VMEM_SHARED((R, D_HALF), jnp.float32),
        ],
    )


def _tc_body(relu, split_in, split_out, x_ref, a_ref, deg_ref, ws_ref, wn_ref, b_ref, o_ref):
    deg = deg_ref[0][:, 0:1] + deg_ref[1][:, 0:1]
    inv = 1.0 / jnp.maximum(deg, 1.0)
    if split_in:
        ys = (jnp.dot(x_ref[0], ws_ref[0:128, :], preferred_element_type=jnp.float32)
              + jnp.dot(x_ref[1], ws_ref[128:256, :], preferred_element_type=jnp.float32))
    else:
        ys = jnp.dot(x_ref[...], ws_ref[...], preferred_element_type=jnp.float32)
    y = (
        ys
        + jnp.dot(a_ref[0] * inv, wn_ref[0:128, :], preferred_element_type=jnp.float32)
        + jnp.dot(a_ref[1] * inv, wn_ref[128:256, :], preferred_element_type=jnp.float32)
        + b_ref[0:1, :]
    )
    if relu:
        y = jnp.maximum(y, 0.0)
    if split_out:
        o_ref[0] = y[:, 0:128]
        o_ref[1] = y[:, 128:256]
    else:
        o_ref[...] = y


def _make_tc_layer(relu, split_in, split_out, bn=2000):
    ngrid = N_NODES // bn
    if split_in:
        x_spec = pl.BlockSpec((NCORES, bn, D_HALF), lambda i: (0, i, 0))
    else:
        x_spec = pl.BlockSpec((bn, D_IN), lambda i: (i, 0))
    if split_out:
        out_shape = jax.ShapeDtypeStruct((NCORES, R, D_HALF), jnp.float32)
        out_spec = pl.BlockSpec((NCORES, bn, D_HALF), lambda i: (0, i, 0))
    else:
        out_shape = jax.ShapeDtypeStruct((N_NODES, D_IN), jnp.float32)
        out_spec = pl.BlockSpec((bn, D_IN), lambda i: (i, 0))
    return pl.pallas_call(
        functools.partial(_tc_body, relu, split_in, split_out),
        grid=(ngrid,),
        in_specs=[
            x_spec,
            pl.BlockSpec((NCORES, bn, D_HALF), lambda i: (0, i, 0)),
            pl.BlockSpec((NCORES, bn, D_HALF), lambda i: (0, i, 0)),
            pl.BlockSpec((D_IN, D_IN), lambda i: (0, 0)),
            pl.BlockSpec((D_IN, D_IN), lambda i: (0, 0)),
            pl.BlockSpec((1, D_IN), lambda i: (0, 0)),
        ],
        out_specs=out_spec,
        out_shape=out_shape,
    )


def kernel(in_feat, edge_index, W_self1, W_neigh1, b1, W_self2, W_neigh2, b2):
    ei = edge_index.astype(jnp.int32)
    src = jnp.concatenate(
        [ei[0], jnp.zeros((E_PAD - N_EDGES,), jnp.int32)])
    dst = jnp.concatenate(
        [ei[1], jnp.full((E_PAD - N_EDGES,), ABSORB, jnp.int32)])
    srcs1 = jnp.stack([2 * src, 2 * src + 1]).reshape(NCORES, NTILES, NCH, CH)
    srcs2 = jnp.stack([src, src + R]).reshape(NCORES, NTILES, NCH, CH)
    dsts = dst.reshape(NTILES, NCH, CH)

    deg16 = _make_sc_deg()(dsts)
    agg1 = _make_sc_agg()(in_feat.reshape(2 * N_NODES, D_HALF), srcs1, dsts)
    h = _make_tc_layer(True, False, True)(
        in_feat, agg1, deg16, W_self1, W_neigh1, b1.reshape(1, D_IN))
    agg2 = _make_sc_agg()(h.reshape(NCORES * R, D_HALF), srcs2, dsts)
    return _make_tc_layer(False, True, False)(
        h, agg2, deg16, W_self2, W_neigh2, b2.reshape(1, D_IN))


# speedup vs baseline: 1.0605x; 1.0605x over previous
import functools

import jax
import jax.numpy as jnp
from jax import lax
from jax.experimental import pallas as pl
from jax.experimental.pallas import tpu as pltpu
from jax.experimental.pallas import tpu_sc as plsc

N_NODES = 10000
N_EDGES = 160000
D_IN = 256
D_HALF = 128

NTILES = 16
NCORES = 2
R = 10240
ROWS_PER_TILE = R // NTILES
ABSORB = N_NODES
CH = 80
NCH = 128
QC = 32
NBUF = 4
E_PAD = NTILES * NCH * CH
ROW_CH = 128


def _sc_agg_body(table, srcs, dsts, agg_out, *scratch):
    src_a, dst_a = scratch[0], scratch[1]
    rows = scratch[2:2 + NBUF]
    gs = scratch[2 + NBUF:2 + 2 * NBUF]
    s0 = scratch[2 + 2 * NBUF]
    agg_s = scratch[3 + 2 * NBUF]
    rows0 = rows[0]
    c = lax.axis_index("c")
    s = lax.axis_index("s")
    base = s * ROWS_PER_TILE

    def _zrow(k, _):
        rows0[k // 8, pl.ds((k % 8) * 16, 16)] = jnp.zeros((16,), jnp.float32)
        return 0

    lax.fori_loop(0, CH * 8, _zrow, 0)
    for k in range(ROWS_PER_TILE // CH):
        pltpu.async_copy(rows0, agg_s.at[pl.ds(base + k * CH, CH)], s0)
    for k in range(ROWS_PER_TILE // CH):
        pltpu.make_async_copy(rows0, agg_s.at[pl.ds(base + k * CH, CH)], s0).wait()

    plsc.subcore_barrier()

    for q in range(NCH // QC):
        pltpu.sync_copy(srcs.at[c, s, pl.ds(q * QC, QC)], src_a)
        pltpu.sync_copy(dsts.at[s, pl.ds(q * QC, QC)], dst_a)
        for b in range(NBUF):
            pltpu.async_copy(table.at[src_a.at[b]], rows[b], gs[b])

        def _grp(k, _):
            jj = NBUF * k
            for b in range(NBUF):
                pltpu.make_async_copy(table.at[src_a.at[jj + b]], rows[b], gs[b]).wait()
                pltpu.async_copy(rows[b], agg_s.at[dst_a.at[jj + b]], s0, add=True)
                pltpu.make_async_copy(rows[b], agg_s.at[dst_a.at[jj + b]], s0).wait()

                def _issue(b=b, jj=jj):
                    pltpu.async_copy(table.at[src_a.at[jj + b + NBUF]], rows[b], gs[b])

                pl.when(k < QC // NBUF - 1)(_issue)
            return 0

        lax.fori_loop(0, QC // NBUF, _grp, 0)
    plsc.subcore_barrier()

    pltpu.sync_copy(agg_s.at[pl.ds(base, ROWS_PER_TILE)],
                    agg_out.at[c, pl.ds(base, ROWS_PER_TILE)])


def _make_sc_agg():
    mesh = plsc.VectorSubcoreMesh(core_axis_name="c", subcore_axis_name="s")
    return pl.kernel(
        _sc_agg_body,
        out_type=jax.ShapeDtypeStruct((NCORES, R, D_HALF), jnp.float32),
        mesh=mesh,
        scratch_types=(
            [pltpu.VMEM((QC, CH), jnp.int32),
             pltpu.VMEM((QC, CH), jnp.int32)]
            + [pltpu.VMEM((CH, D_HALF), jnp.float32) for _ in range(NBUF)]
            + [pltpu.SemaphoreType.DMA for _ in range(NBUF + 1)]
            + [pltpu.VMEM_SHARED((R, D_HALF), jnp.float32)]
        ),
    )


def _sc_deg_body(dsts, deg_out, dst_a, ones_v, sem, deg_s):
    c = lax.axis_index("c")
    s = lax.axis_index("s")
    base = s * ROWS_PER_TILE
    nch = NCH // 2

    def _fill(v):
        def _row(k, _):
            ones_v[k // 8, pl.ds((k % 8) * 16, 16)] = jnp.full((16,), v, jnp.float32)
            return 0

        lax.fori_loop(0, CH * 8, _row, 0)

    _fill(0.0)
    for k in range(ROWS_PER_TILE // CH):
        pltpu.sync_copy(ones_v, deg_s.at[pl.ds(base + k * CH, CH)])
    _fill(1.0)
    pltpu.sync_copy(dsts.at[s, pl.ds(c * nch, nch)], dst_a)
    plsc.subcore_barrier()

    def _chunk(j, _):
        pltpu.async_copy(ones_v, deg_s.at[dst_a.at[j]], sem, add=True)
        pltpu.make_async_copy(ones_v, deg_s.at[dst_a.at[j]], sem).wait()
        return 0

    lax.fori_loop(0, nch, _chunk, 0)
    plsc.subcore_barrier()

    pltpu.sync_copy(deg_s.at[pl.ds(base, ROWS_PER_TILE)],
                    deg_out.at[c, pl.ds(base, ROWS_PER_TILE)])


def _make_sc_deg():
    mesh = plsc.VectorSubcoreMesh(core_axis_name="c", subcore_axis_name="s")
    return pl.kernel(
        _sc_deg_body,
        out_type=jax.ShapeDtypeStruct((NCORES, R, D_HALF), jnp.float32),
        mesh=mesh,
        scratch_types=[
            pltpu.VMEM((NCH // 2, CH), jnp.int32),
            pltpu.VMEM((CH, D_HALF), jnp.float32),
            pltpu.SemaphoreType.DMA,
            pltpu.VMEM_SHARED((R, D_HALF), jnp.float32),
        ],
    )


def _tc_body(relu, split_in, split_out, x_ref, a_ref, deg_ref, ws_ref, wn_ref, b_ref, o_ref):
    deg = deg_ref[0][:, 0:1] + deg_ref[1][:, 0:1]
    inv = 1.0 / jnp.maximum(deg, 1.0)
    if split_in:
        ys = (jnp.dot(x_ref[0], ws_ref[0:128, :], preferred_element_type=jnp.float32)
              + jnp.dot(x_ref[1], ws_ref[128:256, :], preferred_element_type=jnp.float32))
    else:
        ys = jnp.dot(x_ref[...], ws_ref[...], preferred_element_type=jnp.float32)
    y = (
        ys
        + jnp.dot(a_ref[0] * inv, wn_ref[0:128, :], preferred_element_type=jnp.float32)
        + jnp.dot(a_ref[1] * inv, wn_ref[128:256, :], preferred_element_type=jnp.float32)
        + b_ref[0:1, :]
    )
    if relu:
        y = jnp.maximum(y, 0.0)
    if split_out:
        o_ref[0] = y[:, 0:128]
        o_ref[1] = y[:, 128:256]
    else:
        o_ref[...] = y


def _make_tc_layer(relu, split_in, split_out, bn=2000):
    ngrid = N_NODES // bn
    if split_in:
        x_spec = pl.BlockSpec((NCORES, bn, D_HALF), lambda i: (0, i, 0))
    else:
        x_spec = pl.BlockSpec((bn, D_IN), lambda i: (i, 0))
    if split_out:
        out_shape = jax.ShapeDtypeStruct((NCORES, R, D_HALF), jnp.float32)
        out_spec = pl.BlockSpec((NCORES, bn, D_HALF), lambda i: (0, i, 0))
    else:
        out_shape = jax.ShapeDtypeStruct((N_NODES, D_IN), jnp.float32)
        out_spec = pl.BlockSpec((bn, D_IN), lambda i: (i, 0))
    return pl.pallas_call(
        functools.partial(_tc_body, relu, split_in, split_out),
        grid=(ngrid,),
        in_specs=[
            x_spec,
            pl.BlockSpec((NCORES, bn, D_HALF), lambda i: (0, i, 0)),
            pl.BlockSpec((NCORES, bn, D_HALF), lambda i: (0, i, 0)),
            pl.BlockSpec((D_IN, D_IN), lambda i: (0, 0)),
            pl.BlockSpec((D_IN, D_IN), lambda i: (0, 0)),
            pl.BlockSpec((1, D_IN), lambda i: (0, 0)),
        ],
        out_specs=out_spec,
        out_shape=out_shape,
    )


def kernel(in_feat, edge_index, W_self1, W_neigh1, b1, W_self2, W_neigh2, b2):
    ei = edge_index.astype(jnp.int32)
    src = jnp.concatenate(
        [ei[0], jnp.zeros((E_PAD - N_EDGES,), jnp.int32)])
    dst = jnp.concatenate(
        [ei[1], jnp.full((E_PAD - N_EDGES,), ABSORB, jnp.int32)])
    srcs1 = jnp.stack([2 * src, 2 * src + 1]).reshape(NCORES, NTILES, NCH, CH)
    srcs2 = jnp.stack([src, src + R]).reshape(NCORES, NTILES, NCH, CH)
    dsts = dst.reshape(NTILES, NCH, CH)

    deg16 = _make_sc_deg()(dsts)
    agg1 = _make_sc_agg()(in_feat.reshape(2 * N_NODES, D_HALF), srcs1, dsts)
    h = _make_tc_layer(True, False, True)(
        in_feat, agg1, deg16, W_self1, W_neigh1, b1.reshape(1, D_IN))
    agg2 = _make_sc_agg()(h.reshape(NCORES * R, D_HALF), srcs2, dsts)
    return _make_tc_layer(False, True, False)(
        h, agg2, deg16, W_self2, W_neigh2, b2.reshape(1, D_IN))


# speedup vs baseline: 1.0747x; 1.0134x over previous
import functools

import jax
import jax.numpy as jnp
from jax import lax
from jax.experimental import pallas as pl
from jax.experimental.pallas import tpu as pltpu
from jax.experimental.pallas import tpu_sc as plsc

N_NODES = 10000
N_EDGES = 160000
D_IN = 256
D_HALF = 128

NTILES = 16
NCORES = 2
R = 10240
ROWS_PER_TILE = R // NTILES
ABSORB = N_NODES
CH = 80
NCH = 128
QC = 32
NBUF = 4
E_PAD = NTILES * NCH * CH
ROW_CH = 128


def _sc_agg_body(with_deg, table, srcs, dsts, *rest):
    if with_deg:
        agg_out, deg_out = rest[0], rest[1]
        scratch = rest[2:]
    else:
        agg_out, deg_out = rest[0], None
        scratch = rest[1:]
    src_a, dst_a = scratch[0], scratch[1]
    rows = scratch[2:2 + NBUF]
    gs = scratch[2 + NBUF:2 + 2 * NBUF]
    s0 = scratch[2 + 2 * NBUF]
    agg_s = scratch[3 + 2 * NBUF]
    rows0 = rows[0]
    c = lax.axis_index("c")
    s = lax.axis_index("s")
    base = s * ROWS_PER_TILE

    def _zrow(k, _):
        rows0[k // 8, pl.ds((k % 8) * 16, 16)] = jnp.zeros((16,), jnp.float32)
        return 0

    lax.fori_loop(0, CH * 8, _zrow, 0)
    for k in range(ROWS_PER_TILE // CH):
        pltpu.async_copy(rows0, agg_s.at[pl.ds(base + k * CH, CH)], s0)
    for k in range(ROWS_PER_TILE // CH):
        pltpu.make_async_copy(rows0, agg_s.at[pl.ds(base + k * CH, CH)], s0).wait()

    plsc.subcore_barrier()

    for q in range(NCH // QC):
        pltpu.sync_copy(srcs.at[c, s, pl.ds(q * QC, QC)], src_a)
        pltpu.sync_copy(dsts.at[s, pl.ds(q * QC, QC)], dst_a)
        for b in range(NBUF):
            pltpu.async_copy(table.at[src_a.at[b]], rows[b], gs[b])

        def _grp(k, _):
            jj = NBUF * k
            for b in range(NBUF):
                pltpu.make_async_copy(table.at[src_a.at[jj + b]], rows[b], gs[b]).wait()
                pltpu.async_copy(rows[b], agg_s.at[dst_a.at[jj + b]], s0, add=True)
                pltpu.make_async_copy(rows[b], agg_s.at[dst_a.at[jj + b]], s0).wait()

                def _issue(b=b, jj=jj):
                    pltpu.async_copy(table.at[src_a.at[jj + b + NBUF]], rows[b], gs[b])

                pl.when(k < QC // NBUF - 1)(_issue)
            return 0

        lax.fori_loop(0, QC // NBUF, _grp, 0)
    plsc.subcore_barrier()

    pltpu.sync_copy(agg_s.at[pl.ds(base, ROWS_PER_TILE)],
                    agg_out.at[c, pl.ds(base, ROWS_PER_TILE)])

    if deg_out is not None:
        lax.fori_loop(0, CH * 8, _zrow, 0)
        for k in range(ROWS_PER_TILE // CH):
            pltpu.async_copy(rows0, agg_s.at[pl.ds(base + k * CH, CH)], s0)
        for k in range(ROWS_PER_TILE // CH):
            pltpu.make_async_copy(rows0, agg_s.at[pl.ds(base + k * CH, CH)], s0).wait()

        def _orow(k, _):
            rows0[k // 8, pl.ds((k % 8) * 16, 16)] = jnp.full((16,), 1.0, jnp.float32)
            return 0

        lax.fori_loop(0, CH * 8, _orow, 0)
        plsc.subcore_barrier()

        nch = NCH // 2
        for q in range(nch // QC):
            pltpu.sync_copy(dsts.at[s, pl.ds(c * nch + q * QC, QC)], dst_a)

            def _chunk(j, _):
                pltpu.async_copy(rows0, agg_s.at[dst_a.at[j]], s0, add=True)
                pltpu.make_async_copy(rows0, agg_s.at[dst_a.at[j]], s0).wait()
                return 0

            lax.fori_loop(0, QC, _chunk, 0)
        plsc.subcore_barrier()

        pltpu.sync_copy(agg_s.at[pl.ds(base, ROWS_PER_TILE)],
                        deg_out.at[c, pl.ds(base, ROWS_PER_TILE)])


def _make_sc_agg(with_deg=False):
    mesh = plsc.VectorSubcoreMesh(core_axis_name="c", subcore_axis_name="s")
    out = jax.ShapeDtypeStruct((NCORES, R, D_HALF), jnp.float32)
    return pl.kernel(
        functools.partial(_sc_agg_body, with_deg),
        out_type=(out, out) if with_deg else out,
        mesh=mesh,
        scratch_types=(
            [pltpu.VMEM((QC, CH), jnp.int32),
             pltpu.VMEM((QC, CH), jnp.int32)]
            + [pltpu.VMEM((CH, D_HALF), jnp.float32) for _ in range(NBUF)]
            + [pltpu.SemaphoreType.DMA for _ in range(NBUF + 1)]
            + [pltpu.VMEM_SHARED((R, D_HALF), jnp.float32)]
        ),
    )


def _tc_body(relu, split_in, split_out, x_ref, a_ref, deg_ref, ws_ref, wn_ref, b_ref, o_ref):
    deg = deg_ref[0][:, 0:1] + deg_ref[1][:, 0:1]
    inv = 1.0 / jnp.maximum(deg, 1.0)
    if split_in:
        ys = (jnp.dot(x_ref[0], ws_ref[0:128, :], preferred_element_type=jnp.float32)
              + jnp.dot(x_ref[1], ws_ref[128:256, :], preferred_element_type=jnp.float32))
    else:
        ys = jnp.dot(x_ref[...], ws_ref[...], preferred_element_type=jnp.float32)
    y = (
        ys
        + jnp.dot(a_ref[0] * inv, wn_ref[0:128, :], preferred_element_type=jnp.float32)
        + jnp.dot(a_ref[1] * inv, wn_ref[128:256, :], preferred_element_type=jnp.float32)
        + b_ref[0:1, :]
    )
    if relu:
        y = jnp.maximum(y, 0.0)
    if split_out:
        o_ref[0] = y[:, 0:128]
        o_ref[1] = y[:, 128:256]
    else:
        o_ref[...] = y


def _make_tc_layer(relu, split_in, split_out, bn=2000):
    ngrid = N_NODES // bn
    if split_in:
        x_spec = pl.BlockSpec((NCORES, bn, D_HALF), lambda i: (0, i, 0))
    else:
        x_spec = pl.BlockSpec((bn, D_IN), lambda i: (i, 0))
    if split_out:
        out_shape = jax.ShapeDtypeStruct((NCORES, R, D_HALF), jnp.float32)
        out_spec = pl.BlockSpec((NCORES, bn, D_HALF), lambda i: (0, i, 0))
    else:
        out_shape = jax.ShapeDtypeStruct((N_NODES, D_IN), jnp.float32)
        out_spec = pl.BlockSpec((bn, D_IN), lambda i: (i, 0))
    return pl.pallas_call(
        functools.partial(_tc_body, relu, split_in, split_out),
        grid=(ngrid,),
        in_specs=[
            x_spec,
            pl.BlockSpec((NCORES, bn, D_HALF), lambda i: (0, i, 0)),
            pl.BlockSpec((NCORES, bn, D_HALF), lambda i: (0, i, 0)),
            pl.BlockSpec((D_IN, D_IN), lambda i: (0, 0)),
            pl.BlockSpec((D_IN, D_IN), lambda i: (0, 0)),
            pl.BlockSpec((1, D_IN), lambda i: (0, 0)),
        ],
        out_specs=out_spec,
        out_shape=out_shape,
    )


def kernel(in_feat, edge_index, W_self1, W_neigh1, b1, W_self2, W_neigh2, b2):
    ei = edge_index.astype(jnp.int32)
    src = jnp.concatenate(
        [ei[0], jnp.zeros((E_PAD - N_EDGES,), jnp.int32)])
    dst = jnp.concatenate(
        [ei[1], jnp.full((E_PAD - N_EDGES,), ABSORB, jnp.int32)])
    srcs1 = jnp.stack([2 * src, 2 * src + 1]).reshape(NCORES, NTILES, NCH, CH)
    srcs2 = jnp.stack([src, src + R]).reshape(NCORES, NTILES, NCH, CH)
    dsts = dst.reshape(NTILES, NCH, CH)

    agg1, deg16 = _make_sc_agg(True)(
        in_feat.reshape(2 * N_NODES, D_HALF), srcs1, dsts)
    h = _make_tc_layer(True, False, True)(
        in_feat, agg1, deg16, W_self1, W_neigh1, b1.reshape(1, D_IN))
    agg2 = _make_sc_agg()(h.reshape(NCORES * R, D_HALF), srcs2, dsts)
    return _make_tc_layer(False, True, False)(
        h, agg2, deg16, W_self2, W_neigh2, b2.reshape(1, D_IN))
